# Initial kernel scaffold; baseline (speedup 1.0000x reference)
import jax
import jax.numpy as jnp
from jax.experimental import pallas as pl


def kernel(code_ids, position_ids, segment_ids, code_table, pos_table, seg_table, ln_gamma, ln_beta):
    raise NotImplementedError("write your pallas kernel here")


# speedup vs baseline: 1.5037x; 1.5037x over previous
import functools

import jax
import jax.numpy as jnp
from jax import lax
from jax.experimental import pallas as pl
from jax.experimental.pallas import tpu as pltpu
from jax.experimental.pallas import tpu_sc as plsc

B, L, V, D, P, T = 4096, 200, 100000, 64, 512, 2
EPS = 1e-12

NC, NS = 2, 16
NW = NC * NS
N = B * L
ROWS_PER_W = N // NW
CHUNK = 128
NCHUNKS = ROWS_PER_W // CHUNK
NGROUPS = CHUNK // 16


def _ehr_body(cid_hbm, pid_hbm, sid_hbm, code_hbm, pos_hbm, seg_hbm,
              gam_hbm, bet_hbm, out_hbm,
              cid_v, pid_v, sid_v, rows_v, tbuf_v, pos_v, seg_v, gam_v, bet_v,
              sem):
    wid = lax.axis_index("s") * NC + lax.axis_index("c")
    base = wid * ROWS_PER_W

    pltpu.sync_copy(pos_hbm, pos_v)
    pltpu.sync_copy(seg_hbm, seg_v)
    pltpu.sync_copy(gam_hbm, gam_v)
    pltpu.sync_copy(bet_hbm, bet_v)

    iota16 = lax.iota(jnp.int32, 16)
    inv_d = jnp.float32(1.0 / D)

    @pl.loop(0, NCHUNKS)
    def _chunk(c):
        off = base + c * CHUNK
        pltpu.sync_copy(cid_hbm.at[pl.ds(off, CHUNK)], cid_v)
        pltpu.sync_copy(pid_hbm.at[pl.ds(off, CHUNK)], pid_v)
        pltpu.sync_copy(sid_hbm.at[pl.ds(off, CHUNK)], sid_v)
        pltpu.async_copy(code_hbm.at[cid_v], rows_v, sem).wait()

        @pl.loop(0, NGROUPS)
        def _group(g):
            rows16 = iota16 + g * 16
            pids16 = pid_v[pl.ds(g * 16, 16)]
            sids16 = sid_v[pl.ds(g * 16, 16)]

            @pl.loop(0, D, init_carry=(jnp.zeros((16,), jnp.float32),
                                       jnp.zeros((16,), jnp.float32)))
            def _pass1(d, carry):
                s, ss = carry
                dd = jnp.full((16,), d, jnp.int32)
                v = (plsc.load_gather(rows_v, [rows16, dd])
                     + plsc.load_gather(pos_v, [pids16, dd])
                     + plsc.load_gather(seg_v, [sids16, dd]))
                plsc.store_scatter(tbuf_v, [dd, iota16], v)
                return (s + v, ss + v * v)

            s, ss = _pass1
            mu = s * inv_d
            var = ss * inv_d - mu * mu
            x = var + EPS
            i = plsc.bitcast(x, jnp.int32)
            i = jnp.int32(0x5F3759DF) - lax.shift_right_logical(i, 1)
            y = plsc.bitcast(i, jnp.float32)
            y = y * (1.5 - 0.5 * x * y * y)
            y = y * (1.5 - 0.5 * x * y * y)
            y = y * (1.5 - 0.5 * x * y * y)
            rstd = y

            @pl.loop(0, D)
            def _pass2(d):
                dd = jnp.full((16,), d, jnp.int32)
                v = plsc.load_gather(tbuf_v, [dd, iota16])
                zero16 = jnp.zeros((16,), jnp.int32)
                gv = plsc.load_gather(gam_v, [dd, zero16])
                bv = plsc.load_gather(bet_v, [dd, zero16])
                out = (v - mu) * rstd * gv + bv
                plsc.store_scatter(rows_v, [rows16, dd], out)

        pltpu.sync_copy(rows_v, out_hbm.at[pl.ds(off, CHUNK)])


@jax.jit
def _ehr(cid, pid, sid, code_table, pos_table, seg_table, gam, bet):
    mesh = plsc.VectorSubcoreMesh(core_axis_name="c", subcore_axis_name="s")
    return pl.kernel(
        _ehr_body,
        out_type=jax.ShapeDtypeStruct((N, D), jnp.float32),
        mesh=mesh,
        compiler_params=pltpu.CompilerParams(
            needs_layout_passes=False, use_tc_tiling_on_sc=False),
        scratch_types=[
            pltpu.VMEM((CHUNK,), jnp.int32),
            pltpu.VMEM((CHUNK,), jnp.int32),
            pltpu.VMEM((CHUNK,), jnp.int32),
            pltpu.VMEM((CHUNK, D), jnp.float32),
            pltpu.VMEM((D, 16), jnp.float32),
            pltpu.VMEM((P, D), jnp.float32),
            pltpu.VMEM((T, D), jnp.float32),
            pltpu.VMEM((D, 1), jnp.float32),
            pltpu.VMEM((D, 1), jnp.float32),
            pltpu.SemaphoreType.DMA,
        ],
    )(cid, pid, sid, code_table, pos_table, seg_table, gam, bet)


def kernel(code_ids, position_ids, segment_ids, code_table, pos_table,
           seg_table, ln_gamma, ln_beta):
    cid = code_ids.reshape(-1).astype(jnp.int32)
    pid = position_ids.reshape(-1).astype(jnp.int32)
    sid = segment_ids.reshape(-1).astype(jnp.int32)
    out = _ehr(cid, pid, sid, code_table, pos_table, seg_table,
               ln_gamma.reshape(D, 1), ln_beta.reshape(D, 1))
    return out.reshape(code_ids.shape + (D,))


# speedup vs baseline: 1.5074x; 1.0024x over previous
import functools

import jax
import jax.numpy as jnp
from jax import lax
from jax.experimental import pallas as pl
from jax.experimental.pallas import tpu as pltpu
from jax.experimental.pallas import tpu_sc as plsc

B, L, V, D, P, T = 4096, 200, 100000, 64, 512, 2
EPS = 1e-12

NC, NS = 2, 16
NW = NC * NS
N = B * L
ROWS_PER_W = N // NW
CHUNK = 128
NCHUNKS = ROWS_PER_W // CHUNK
NGROUPS = CHUNK // 16


def _ehr_body(cid_hbm, pid_hbm, sid_hbm, code_hbm, pos_hbm, seg_hbm,
              gam_hbm, bet_hbm, out_hbm,
              cid_v, pid_v, sid_v, rows_v, tbuf_v, pos_v, seg_v, gam_v, bet_v,
              sem):
    wid = lax.axis_index("s") * NC + lax.axis_index("c")
    base = wid * ROWS_PER_W

    pltpu.sync_copy(pos_hbm, pos_v)
    pltpu.sync_copy(seg_hbm, seg_v)
    pltpu.sync_copy(gam_hbm, gam_v)
    pltpu.sync_copy(bet_hbm, bet_v)

    iota16 = lax.iota(jnp.int32, 16)
    inv_d = jnp.float32(1.0 / D)

    @pl.loop(0, NCHUNKS)
    def _chunk(c):
        off = base + c * CHUNK
        pltpu.sync_copy(cid_hbm.at[pl.ds(off, CHUNK)], cid_v)
        pltpu.sync_copy(pid_hbm.at[pl.ds(off, CHUNK)], pid_v)
        pltpu.sync_copy(sid_hbm.at[pl.ds(off, CHUNK)], sid_v)
        pltpu.async_copy(code_hbm.at[cid_v], rows_v, sem).wait()

        @pl.loop(0, NGROUPS)
        def _group(g):
            rows16 = iota16 + g * 16
            pids16 = pid_v[pl.ds(g * 16, 16)]
            sids16 = sid_v[pl.ds(g * 16, 16)]

            @pl.loop(0, D, init_carry=(jnp.zeros((16,), jnp.float32),
                                       jnp.zeros((16,), jnp.float32)),
                     unroll=16)
            def _pass1(d, carry):
                s, ss = carry
                dd = jnp.full((16,), d, jnp.int32)
                v = (plsc.load_gather(rows_v, [rows16, dd])
                     + plsc.load_gather(pos_v, [pids16, dd])
                     + plsc.load_gather(seg_v, [sids16, dd]))
                plsc.store_scatter(tbuf_v, [dd, iota16], v)
                return (s + v, ss + v * v)

            s, ss = _pass1
            mu = s * inv_d
            var = ss * inv_d - mu * mu
            x = var + EPS
            i = plsc.bitcast(x, jnp.int32)
            i = jnp.int32(0x5F3759DF) - lax.shift_right_logical(i, 1)
            y = plsc.bitcast(i, jnp.float32)
            y = y * (1.5 - 0.5 * x * y * y)
            y = y * (1.5 - 0.5 * x * y * y)
            y = y * (1.5 - 0.5 * x * y * y)
            rstd = y

            @pl.loop(0, D, unroll=16)
            def _pass2(d):
                dd = jnp.full((16,), d, jnp.int32)
                v = plsc.load_gather(tbuf_v, [dd, iota16])
                zero16 = jnp.zeros((16,), jnp.int32)
                gv = plsc.load_gather(gam_v, [dd, zero16])
                bv = plsc.load_gather(bet_v, [dd, zero16])
                out = (v - mu) * rstd * gv + bv
                plsc.store_scatter(rows_v, [rows16, dd], out)

        pltpu.sync_copy(rows_v, out_hbm.at[pl.ds(off, CHUNK)])


@jax.jit
def _ehr(cid, pid, sid, code_table, pos_table, seg_table, gam, bet):
    mesh = plsc.VectorSubcoreMesh(core_axis_name="c", subcore_axis_name="s")
    return pl.kernel(
        _ehr_body,
        out_type=jax.ShapeDtypeStruct((N, D), jnp.float32),
        mesh=mesh,
        compiler_params=pltpu.CompilerParams(
            needs_layout_passes=False, use_tc_tiling_on_sc=False),
        scratch_types=[
            pltpu.VMEM((CHUNK,), jnp.int32),
            pltpu.VMEM((CHUNK,), jnp.int32),
            pltpu.VMEM((CHUNK,), jnp.int32),
            pltpu.VMEM((CHUNK, D), jnp.float32),
            pltpu.VMEM((D, 16), jnp.float32),
            pltpu.VMEM((P, D), jnp.float32),
            pltpu.VMEM((T, D), jnp.float32),
            pltpu.VMEM((D, 1), jnp.float32),
            pltpu.VMEM((D, 1), jnp.float32),
            pltpu.SemaphoreType.DMA,
        ],
    )(cid, pid, sid, code_table, pos_table, seg_table, gam, bet)


def kernel(code_ids, position_ids, segment_ids, code_table, pos_table,
           seg_table, ln_gamma, ln_beta):
    cid = code_ids.reshape(-1).astype(jnp.int32)
    pid = position_ids.reshape(-1).astype(jnp.int32)
    sid = segment_ids.reshape(-1).astype(jnp.int32)
    out = _ehr(cid, pid, sid, code_table, pos_table, seg_table,
               ln_gamma.reshape(D, 1), ln_beta.reshape(D, 1))
    return out.reshape(code_ids.shape + (D,))


# speedup vs baseline: 1.6144x; 1.0710x over previous
import functools

import jax
import jax.numpy as jnp
from jax import lax
from jax.experimental import pallas as pl
from jax.experimental.pallas import tpu as pltpu
from jax.experimental.pallas import tpu_sc as plsc

B, L, V, D, P, T = 4096, 200, 100000, 64, 512, 2
EPS = 1e-12

NC, NS = 2, 16
NW = NC * NS
N = B * L
ROWS_PER_W = N // NW
CHUNK = 128
NCHUNKS = ROWS_PER_W // CHUNK
NGROUPS = CHUNK // 16


def _ehr_body(ids_hbm, code_hbm, pos_hbm, seg_hbm, gam_hbm, bet_hbm, out_hbm,
              ids_v0, ids_v1, rows_v0, rows_v1, out_v0, out_v1, tbuf_v,
              pos_v, seg_v, gam_v, bet_v,
              ids_s0, ids_s1, g_s0, g_s1, o_s0, o_s1):
    wid = lax.axis_index("s") * NC + lax.axis_index("c")
    cbase = wid * NCHUNKS

    ids_v = (ids_v0, ids_v1)
    rows_v = (rows_v0, rows_v1)
    out_v = (out_v0, out_v1)
    ids_s = (ids_s0, ids_s1)
    g_s = (g_s0, g_s1)
    o_s = (o_s0, o_s1)

    pltpu.sync_copy(pos_hbm, pos_v)
    pltpu.sync_copy(seg_hbm, seg_v)
    pltpu.sync_copy(gam_hbm, gam_v)
    pltpu.sync_copy(bet_hbm, bet_v)

    iota16 = lax.iota(jnp.int32, 16)
    inv_d = jnp.float32(1.0 / D)

    def ids_copy(c, p):
        return pltpu.make_async_copy(ids_hbm.at[cbase + c], ids_v[p],
                                     ids_s[p])

    def gather_copy(c, p):
        return pltpu.make_async_copy(code_hbm.at[ids_v[p].at[0]],
                                     rows_v[p], g_s[p])

    def out_copy(c, p):
        off = (cbase + c) * CHUNK
        return pltpu.make_async_copy(out_v[p], out_hbm.at[pl.ds(off, CHUNK)],
                                     o_s[p])

    def compute(c, p):
        rv, ov, iv = rows_v[p], out_v[p], ids_v[p]

        @pl.loop(0, NGROUPS)
        def _group(g):
            rows16 = iota16 + g * 16
            pids16 = iv[1, pl.ds(g * 16, 16)]
            sids16 = iv[2, pl.ds(g * 16, 16)]

            @pl.loop(0, D, init_carry=(jnp.zeros((16,), jnp.float32),
                                       jnp.zeros((16,), jnp.float32)),
                     unroll=16)
            def _pass1(d, carry):
                s, ss = carry
                dd = jnp.full((16,), d, jnp.int32)
                v = (plsc.load_gather(rv, [rows16, dd])
                     + plsc.load_gather(pos_v, [pids16, dd])
                     + plsc.load_gather(seg_v, [sids16, dd]))
                plsc.store_scatter(tbuf_v, [dd, iota16], v)
                return (s + v, ss + v * v)

            s, ss = _pass1
            mu = s * inv_d
            var = ss * inv_d - mu * mu
            x = var + EPS
            i = plsc.bitcast(x, jnp.int32)
            i = jnp.int32(0x5F3759DF) - lax.shift_right_logical(i, 1)
            y = plsc.bitcast(i, jnp.float32)
            y = y * (1.5 - 0.5 * x * y * y)
            y = y * (1.5 - 0.5 * x * y * y)
            y = y * (1.5 - 0.5 * x * y * y)
            rstd = y

            @pl.loop(0, D, unroll=16)
            def _pass2(d):
                dd = jnp.full((16,), d, jnp.int32)
                v = plsc.load_gather(tbuf_v, [dd, iota16])
                zero16 = jnp.zeros((16,), jnp.int32)
                gv = plsc.load_gather(gam_v, [dd, zero16])
                bv = plsc.load_gather(bet_v, [dd, zero16])
                out = (v - mu) * rstd * gv + bv
                plsc.store_scatter(ov, [rows16, dd], out)

    ids_copy(0, 0).start()
    ids_copy(1, 1).start()
    ids_copy(0, 0).wait()
    gather_copy(0, 0).start()

    @pl.loop(0, NCHUNKS // 2)
    def _chunk2(c2):
        for p in (0, 1):
            c = c2 * 2 + p
            q = 1 - p

            @pl.when(c + 1 < NCHUNKS)
            def _():
                ids_copy(c + 1, q).wait()
                gather_copy(c + 1, q).start()

            gather_copy(c, p).wait()

            @pl.when(c >= 2)
            def _():
                out_copy(c - 2, p).wait()

            compute(c, p)

            @pl.when(c + 2 < NCHUNKS)
            def _():
                ids_copy(c + 2, p).start()

            out_copy(c, p).start()

    out_copy(NCHUNKS - 2, 0).wait()
    out_copy(NCHUNKS - 1, 1).wait()


@jax.jit
def _ehr(ids3, code_table, pos_table, seg_table, gam, bet):
    mesh = plsc.VectorSubcoreMesh(core_axis_name="c", subcore_axis_name="s")
    return pl.kernel(
        _ehr_body,
        out_type=jax.ShapeDtypeStruct((N, D), jnp.float32),
        mesh=mesh,
        compiler_params=pltpu.CompilerParams(
            needs_layout_passes=False, use_tc_tiling_on_sc=False),
        scratch_types=[
            pltpu.VMEM((3, CHUNK), jnp.int32),
            pltpu.VMEM((3, CHUNK), jnp.int32),
            pltpu.VMEM((CHUNK, D), jnp.float32),
            pltpu.VMEM((CHUNK, D), jnp.float32),
            pltpu.VMEM((CHUNK, D), jnp.float32),
            pltpu.VMEM((CHUNK, D), jnp.float32),
            pltpu.VMEM((D, 16), jnp.float32),
            pltpu.VMEM((P, D), jnp.float32),
            pltpu.VMEM((T, D), jnp.float32),
            pltpu.VMEM((D, 1), jnp.float32),
            pltpu.VMEM((D, 1), jnp.float32),
            pltpu.SemaphoreType.DMA,
            pltpu.SemaphoreType.DMA,
            pltpu.SemaphoreType.DMA,
            pltpu.SemaphoreType.DMA,
            pltpu.SemaphoreType.DMA,
            pltpu.SemaphoreType.DMA,
        ],
    )(ids3, code_table, pos_table, seg_table, gam, bet)


def kernel(code_ids, position_ids, segment_ids, code_table, pos_table,
           seg_table, ln_gamma, ln_beta):
    nch = N // CHUNK
    ids3 = jnp.stack([
        code_ids.reshape(nch, CHUNK).astype(jnp.int32),
        position_ids.reshape(nch, CHUNK).astype(jnp.int32),
        segment_ids.reshape(nch, CHUNK).astype(jnp.int32),
    ], axis=1)
    out = _ehr(ids3, code_table, pos_table, seg_table,
               ln_gamma.reshape(D, 1), ln_beta.reshape(D, 1))
    return out.reshape(code_ids.shape + (D,))


# speedup vs baseline: 4.2514x; 2.6335x over previous
import functools

import jax
import jax.numpy as jnp
from jax import lax
from jax.experimental import pallas as pl
from jax.experimental.pallas import tpu as pltpu
from jax.experimental.pallas import tpu_sc as plsc

B, L, V, D, P, T = 4096, 200, 100000, 64, 512, 2
EPS = 1e-12

NC, NS = 2, 16
NW = NC * NS
N = B * L
ROWS_PER_W = N // NW
CHUNK = 128
NCHUNKS = ROWS_PER_W // CHUNK
NGROUPS = CHUNK // 16


def _ehr_body(cid_hbm, ps_hbm, code_hbm, pos_hbm, seg_hbm, gam_hbm, bet_hbm,
              out_hbm,
              cid_v0, cid_v1, ps_v0, ps_v1, rows_v0, rows_v1, out_v0, out_v1,
              pos_v, seg_v, gam_v, bet_v,
              ids_s0, ids_s1, g_s0, g_s1, o_s0, o_s1):
    wid = lax.axis_index("s") * NC + lax.axis_index("c")
    cbase = wid * NCHUNKS

    cid_v = (cid_v0, cid_v1)
    ps_v = (ps_v0, ps_v1)
    rows_v = (rows_v0, rows_v1)
    out_v = (out_v0, out_v1)
    ids_s = (ids_s0, ids_s1)
    g_s = (g_s0, g_s1)
    o_s = (o_s0, o_s1)

    pltpu.sync_copy(pos_hbm, pos_v)
    pltpu.sync_copy(seg_hbm, seg_v)
    pltpu.sync_copy(gam_hbm, gam_v)
    pltpu.sync_copy(bet_hbm, bet_v)

    inv_d = jnp.float32(1.0 / D)

    def cid_copy(c, p):
        return pltpu.make_async_copy(cid_hbm.at[cbase + c], cid_v[p],
                                     ids_s[p])

    def ps_copy(c, p):
        return pltpu.make_async_copy(ps_hbm.at[cbase + c], ps_v[p], ids_s[p])

    def ids_start(c, p):
        cid_copy(c, p).start()
        ps_copy(c, p).start()

    def ids_wait(c, p):
        cid_copy(c, p).wait()
        ps_copy(c, p).wait()

    def gather_copy(c, p):
        return pltpu.make_async_copy(code_hbm.at[cid_v[p]], rows_v[p], g_s[p])

    def out_copy(c, p):
        off = (cbase + c) * CHUNK
        return pltpu.make_async_copy(out_v[p], out_hbm.at[pl.ds(off, CHUNK)],
                                     o_s[p])

    gam = [gam_v[pl.ds(k * 16, 16)] for k in range(4)]
    bet = [bet_v[pl.ds(k * 16, 16)] for k in range(4)]

    def compute(c, p):
        rv, ov, iv = rows_v[p], out_v[p], ps_v[p]

        @pl.loop(0, NGROUPS)
        def _group(g):
            pvec = iv[0, pl.ds(g * 16, 16)]
            svec = iv[1, pl.ds(g * 16, 16)]
            for j in range(16):
                r = g * 16 + j
                pid = pvec[j]
                sid = svec[j]
                v = [rv[r, pl.ds(k * 16, 16)]
                     + pos_v[pid, pl.ds(k * 16, 16)]
                     + seg_v[sid, pl.ds(k * 16, 16)]
                     for k in range(4)]
                s = jnp.sum((v[0] + v[1]) + (v[2] + v[3]))
                ss = jnp.sum((v[0] * v[0] + v[1] * v[1])
                             + (v[2] * v[2] + v[3] * v[3]))
                mu = s * inv_d
                var = ss * inv_d - mu * mu
                x = jnp.full((16,), var + EPS, jnp.float32)
                i = plsc.bitcast(x, jnp.int32)
                i = jnp.int32(0x5F3759DF) - lax.shift_right_logical(i, 1)
                y = plsc.bitcast(i, jnp.float32)
                y = y * (1.5 - 0.5 * x * y * y)
                y = y * (1.5 - 0.5 * x * y * y)
                y = y * (1.5 - 0.5 * x * y * y)
                rstd = y
                for k in range(4):
                    ov[r, pl.ds(k * 16, 16)] = ((v[k] - mu) * rstd * gam[k]
                                                + bet[k])

    ids_start(0, 0)
    ids_start(1, 1)
    ids_wait(0, 0)
    gather_copy(0, 0).start()

    @pl.loop(0, NCHUNKS // 2)
    def _chunk2(c2):
        for p in (0, 1):
            c = c2 * 2 + p
            q = 1 - p

            @pl.when(c + 1 < NCHUNKS)
            def _():
                ids_wait(c + 1, q)
                gather_copy(c + 1, q).start()

            gather_copy(c, p).wait()

            @pl.when(c >= 2)
            def _():
                out_copy(c - 2, p).wait()

            compute(c, p)

            @pl.when(c + 2 < NCHUNKS)
            def _():
                ids_start(c + 2, p)

            out_copy(c, p).start()

    out_copy(NCHUNKS - 2, 0).wait()
    out_copy(NCHUNKS - 1, 1).wait()


@jax.jit
def _ehr(cid3, ps3, code_table, pos_table, seg_table, gam, bet):
    mesh = plsc.VectorSubcoreMesh(core_axis_name="c", subcore_axis_name="s")
    return pl.kernel(
        _ehr_body,
        out_type=jax.ShapeDtypeStruct((N, D), jnp.float32),
        mesh=mesh,
        compiler_params=pltpu.CompilerParams(
            needs_layout_passes=False, use_tc_tiling_on_sc=False),
        scratch_types=[
            pltpu.VMEM((CHUNK,), jnp.int32),
            pltpu.VMEM((CHUNK,), jnp.int32),
            pltpu.VMEM((2, CHUNK), jnp.int32),
            pltpu.VMEM((2, CHUNK), jnp.int32),
            pltpu.VMEM((CHUNK, D), jnp.float32),
            pltpu.VMEM((CHUNK, D), jnp.float32),
            pltpu.VMEM((CHUNK, D), jnp.float32),
            pltpu.VMEM((CHUNK, D), jnp.float32),
            pltpu.VMEM((P, D), jnp.float32),
            pltpu.VMEM((T, D), jnp.float32),
            pltpu.VMEM((D,), jnp.float32),
            pltpu.VMEM((D,), jnp.float32),
            pltpu.SemaphoreType.DMA,
            pltpu.SemaphoreType.DMA,
            pltpu.SemaphoreType.DMA,
            pltpu.SemaphoreType.DMA,
            pltpu.SemaphoreType.DMA,
            pltpu.SemaphoreType.DMA,
        ],
    )(cid3, ps3, code_table, pos_table, seg_table, gam, bet)


def kernel(code_ids, position_ids, segment_ids, code_table, pos_table,
           seg_table, ln_gamma, ln_beta):
    nch = N // CHUNK
    cid3 = code_ids.reshape(nch, CHUNK).astype(jnp.int32)
    ps3 = jnp.stack([
        position_ids.reshape(nch, CHUNK).astype(jnp.int32),
        segment_ids.reshape(nch, CHUNK).astype(jnp.int32),
    ], axis=1)
    out = _ehr(cid3, ps3, code_table, pos_table, seg_table, ln_gamma, ln_beta)
    return out.reshape(code_ids.shape + (D,))


# speedup vs baseline: 8.3784x; 1.9707x over previous
import functools

import jax
import jax.numpy as jnp
from jax import lax
from jax.experimental import pallas as pl
from jax.experimental.pallas import tpu as pltpu
from jax.experimental.pallas import tpu_sc as plsc

B, L, V, D, P, T = 4096, 200, 100000, 64, 512, 2
EPS = 1e-12

NC, NS = 2, 16
NW = NC * NS
N = B * L
ROWS_PER_W = N // NW
CHUNK = 128
NCHUNKS = ROWS_PER_W // CHUNK
NGROUPS = CHUNK // 16


def _ehr_body(cid_hbm, cmb_hbm, code_hbm, comb_hbm, gam_hbm, bet_hbm,
              out_hbm,
              cid_v0, cid_v1, cmb_v0, cmb_v1, rows_v0, rows_v1, out_v0, out_v1,
              comb_v, gam_v, bet_v,
              ids_s0, ids_s1, g_s0, g_s1, o_s0, o_s1):
    wid = lax.axis_index("s") * NC + lax.axis_index("c")
    cbase = wid * NCHUNKS

    cid_v = (cid_v0, cid_v1)
    cmb_v = (cmb_v0, cmb_v1)
    rows_v = (rows_v0, rows_v1)
    out_v = (out_v0, out_v1)
    ids_s = (ids_s0, ids_s1)
    g_s = (g_s0, g_s1)
    o_s = (o_s0, o_s1)

    pltpu.sync_copy(comb_hbm, comb_v)
    pltpu.sync_copy(gam_hbm, gam_v)
    pltpu.sync_copy(bet_hbm, bet_v)

    inv_d = jnp.float32(1.0 / D)

    def cid_copy(c, p):
        return pltpu.make_async_copy(cid_hbm.at[cbase + c], cid_v[p],
                                     ids_s[p])

    def cmb_copy(c, p):
        return pltpu.make_async_copy(cmb_hbm.at[cbase + c], cmb_v[p],
                                     ids_s[p])

    def ids_start(c, p):
        cid_copy(c, p).start()
        cmb_copy(c, p).start()

    def ids_wait(c, p):
        cid_copy(c, p).wait()
        cmb_copy(c, p).wait()

    def gather_copy(c, p):
        return pltpu.make_async_copy(code_hbm.at[cid_v[p]], rows_v[p], g_s[p])

    def out_copy(c, p):
        off = (cbase + c) * CHUNK
        return pltpu.make_async_copy(out_v[p], out_hbm.at[pl.ds(off, CHUNK)],
                                     o_s[p])

    gam = [gam_v[pl.ds(k * 16, 16)] for k in range(4)]
    bet = [bet_v[pl.ds(k * 16, 16)] for k in range(4)]
    iota16 = lax.iota(jnp.int32, 16)
    idx15 = jnp.full((16,), 15, jnp.int32)
    lane_idx = [jnp.full((16,), j, jnp.int32) for j in range(16)]

    def compute(c, p):
        rv, ov, iv = rows_v[p], out_v[p], cmb_v[p]

        @pl.loop(0, NGROUPS)
        def _group(g):
            cvec = iv[pl.ds(g * 16, 16)]
            s_l = jnp.zeros((16,), jnp.float32)
            q_l = jnp.zeros((16,), jnp.float32)
            for j in range(16):
                r = g * 16 + j
                cm = cvec[j]
                v = [rv[r, pl.ds(k * 16, 16)] + comb_v[cm, pl.ds(k * 16, 16)]
                     for k in range(4)]
                for k in range(4):
                    ov[r, pl.ds(k * 16, 16)] = v[k]
                t = (v[0] + v[1]) + (v[2] + v[3])
                q = (v[0] * v[0] + v[1] * v[1]) + (v[2] * v[2] + v[3] * v[3])
                ts = jnp.take_along_axis(plsc.cumsum(t), idx15, axis=0)
                qs = jnp.take_along_axis(plsc.cumsum(q), idx15, axis=0)
                mask = iota16 == j
                s_l = jnp.where(mask, ts, s_l)
                q_l = jnp.where(mask, qs, q_l)
            mu_l = s_l * inv_d
            var_l = q_l * inv_d - mu_l * mu_l
            x = var_l + EPS
            i = plsc.bitcast(x, jnp.int32)
            i = jnp.int32(0x5F3759DF) - lax.shift_right_logical(i, 1)
            y = plsc.bitcast(i, jnp.float32)
            y = y * (1.5 - 0.5 * x * y * y)
            y = y * (1.5 - 0.5 * x * y * y)
            y = y * (1.5 - 0.5 * x * y * y)
            rstd_l = y
            for j in range(16):
                r = g * 16 + j
                muj = jnp.take_along_axis(mu_l, lane_idx[j], axis=0)
                rsj = jnp.take_along_axis(rstd_l, lane_idx[j], axis=0)
                for k in range(4):
                    vk = ov[r, pl.ds(k * 16, 16)]
                    ov[r, pl.ds(k * 16, 16)] = ((vk - muj) * rsj * gam[k]
                                                + bet[k])

    ids_start(0, 0)
    ids_start(1, 1)
    ids_wait(0, 0)
    gather_copy(0, 0).start()

    @pl.loop(0, NCHUNKS // 2)
    def _chunk2(c2):
        for p in (0, 1):
            c = c2 * 2 + p
            q = 1 - p

            @pl.when(c + 1 < NCHUNKS)
            def _():
                ids_wait(c + 1, q)
                gather_copy(c + 1, q).start()

            gather_copy(c, p).wait()

            @pl.when(c >= 2)
            def _():
                out_copy(c - 2, p).wait()

            compute(c, p)

            @pl.when(c + 2 < NCHUNKS)
            def _():
                ids_start(c + 2, p)

            out_copy(c, p).start()

    out_copy(NCHUNKS - 2, 0).wait()
    out_copy(NCHUNKS - 1, 1).wait()


@jax.jit
def _ehr(cid3, cmb3, code_table, comb_table, gam, bet):
    mesh = plsc.VectorSubcoreMesh(core_axis_name="c", subcore_axis_name="s")
    return pl.kernel(
        _ehr_body,
        out_type=jax.ShapeDtypeStruct((N, D), jnp.float32),
        mesh=mesh,
        compiler_params=pltpu.CompilerParams(
            needs_layout_passes=False, use_tc_tiling_on_sc=False),
        scratch_types=[
            pltpu.VMEM((CHUNK,), jnp.int32),
            pltpu.VMEM((CHUNK,), jnp.int32),
            pltpu.VMEM((CHUNK,), jnp.int32),
            pltpu.VMEM((CHUNK,), jnp.int32),
            pltpu.VMEM((CHUNK, D), jnp.float32),
            pltpu.VMEM((CHUNK, D), jnp.float32),
            pltpu.VMEM((CHUNK, D), jnp.float32),
            pltpu.VMEM((CHUNK, D), jnp.float32),
            pltpu.VMEM((P * T, D), jnp.float32),
            pltpu.VMEM((D,), jnp.float32),
            pltpu.VMEM((D,), jnp.float32),
            pltpu.SemaphoreType.DMA,
            pltpu.SemaphoreType.DMA,
            pltpu.SemaphoreType.DMA,
            pltpu.SemaphoreType.DMA,
            pltpu.SemaphoreType.DMA,
            pltpu.SemaphoreType.DMA,
        ],
    )(cid3, cmb3, code_table, comb_table, gam, bet)


def kernel(code_ids, position_ids, segment_ids, code_table, pos_table,
           seg_table, ln_gamma, ln_beta):
    nch = N // CHUNK
    cid3 = code_ids.reshape(nch, CHUNK).astype(jnp.int32)
    cmb3 = (position_ids.astype(jnp.int32) * T
            + segment_ids.astype(jnp.int32)).reshape(nch, CHUNK)
    comb_table = (pos_table[:, None, :] + seg_table[None, :, :]).reshape(
        P * T, D)
    out = _ehr(cid3, cmb3, code_table, comb_table, ln_gamma, ln_beta)
    return out.reshape(code_ids.shape + (D,))
